# baseline (speedup 1.0000x reference)
import functools

import jax
import jax.numpy as jnp
from jax import lax
from jax.experimental import pallas as pl
from jax.experimental.pallas import tpu as pltpu
from jax.experimental.pallas import tpu_sc as plsc

NC, NS = 2, 16
NW = NC * NS
EPS = 1e-8
BN_EPS = 1e-5


def _to_bf16_linear(table, rows_per_block):
    n, e = table.shape

    def body(t_ref, o_ref):
        o_ref[...] = t_ref[...].astype(jnp.bfloat16)

    return pl.pallas_call(
        body,
        grid=(n // rows_per_block,),
        in_specs=[pl.BlockSpec((rows_per_block, e), lambda i: (i, 0))],
        out_specs=pl.BlockSpec((rows_per_block, e), lambda i: (i, 0)),
        out_shape=jax.ShapeDtypeStruct((n, e), jnp.bfloat16),
    )(table)


def _gather_rows(tab16, seqs):
    b, l = seqs.shape
    n, e = tab16.shape
    bpw = b // NW
    nbuf = 4
    mesh = plsc.VectorSubcoreMesh(core_axis_name="c", subcore_axis_name="s")

    @functools.partial(
        pl.kernel,
        out_type=jax.ShapeDtypeStruct((b * l, e), jnp.bfloat16),
        mesh=mesh,
        scratch_types=[
            pltpu.VMEM((bpw, l), jnp.int32),
            pltpu.VMEM((nbuf, l, e), jnp.bfloat16),
            pltpu.SemaphoreType.DMA((nbuf,)),
            pltpu.SemaphoreType.DMA((nbuf,)),
        ],
        compiler_params=pltpu.CompilerParams(use_tc_tiling_on_sc=False),
    )
    def kern(tab_hbm, seq_hbm, out_hbm, idx_v, rows_v, gsems, osems):
        wid = lax.axis_index("s") * NC + lax.axis_index("c")
        base = wid * bpw
        pltpu.sync_copy(seq_hbm.at[pl.ds(base, bpw)], idx_v)

        def fire(g, slot):
            pltpu.async_copy(
                tab_hbm.at[idx_v.at[g]], rows_v.at[slot], gsems.at[slot])

        def gwait(g, slot):
            pltpu.make_async_copy(
                tab_hbm.at[idx_v.at[g]], rows_v.at[slot], gsems.at[slot]).wait()

        def odst(g):
            return out_hbm.at[pl.ds((base + g) * l, l)]

        def owait(g, slot):
            pltpu.make_async_copy(
                rows_v.at[slot], odst(g), osems.at[slot]).wait()

        for p in range(nbuf - 1):
            fire(p, p)

        def body(g, carry):
            slot = lax.rem(g, nbuf)

            @pl.when(g + nbuf - 1 < bpw)
            def _():
                ns = lax.rem(g + nbuf - 1, nbuf)

                @pl.when(g >= 1)
                def _():
                    owait(g - 1, ns)

                fire(g + nbuf - 1, ns)

            gwait(g, slot)
            pltpu.async_copy(rows_v.at[slot], odst(g), osems.at[slot])
            return carry

        lax.fori_loop(0, bpw, body, 0)
        for t in range(nbuf):
            g = bpw - nbuf + t
            owait(g, lax.rem(g, nbuf))

    return kern(tab16, seqs)


def _emb_scores(emb2, w, bb, l2):
    ee = emb2.shape[1]
    b = emb2.shape[0] // l2
    g = b // bb

    def body(e_ref, w_ref, se_ref, so_ref, me_ref, mo_ref, p_ref):
        e = e_ref[...].astype(jnp.float32).reshape(bb, l2, ee)
        lo = e[:, :, :64]
        hi = e[:, :, 64:]
        wv = w_ref[0]
        dn = (((2,), (0,)), ((), ()))
        se = lax.dot_general(lo, wv, dn, preferred_element_type=jnp.float32)
        so = lax.dot_general(hi, wv, dn, preferred_element_type=jnp.float32)
        se_ref[...] = se
        so_ref[...] = so
        me_ref[...] = jnp.any(lo != 0.0, axis=2).astype(jnp.float32)
        mo_ref[...] = jnp.any(hi != 0.0, axis=2).astype(jnp.float32)
        parts = jnp.concatenate(
            [jnp.sum(se, axis=0, keepdims=True),
             jnp.sum(se * se, axis=0, keepdims=True),
             jnp.sum(so, axis=0, keepdims=True),
             jnp.sum(so * so, axis=0, keepdims=True)], axis=0)
        p_ref[...] = parts.reshape(1, 4, l2)

    return pl.pallas_call(
        body,
        grid=(g,),
        in_specs=[
            pl.BlockSpec((bb * l2, ee), lambda i: (i, 0)),
            pl.BlockSpec((1, 64), lambda i: (0, 0)),
        ],
        out_specs=[
            pl.BlockSpec((bb, l2), lambda i: (i, 0)),
            pl.BlockSpec((bb, l2), lambda i: (i, 0)),
            pl.BlockSpec((bb, l2), lambda i: (i, 0)),
            pl.BlockSpec((bb, l2), lambda i: (i, 0)),
            pl.BlockSpec((1, 4, l2), lambda i: (i, 0, 0)),
        ],
        out_shape=[
            jax.ShapeDtypeStruct((b, l2), jnp.float32),
            jax.ShapeDtypeStruct((b, l2), jnp.float32),
            jax.ShapeDtypeStruct((b, l2), jnp.float32),
            jax.ShapeDtypeStruct((b, l2), jnp.float32),
            jax.ShapeDtypeStruct((g, 4, l2), jnp.float32),
        ],
    )(emb2, w.reshape(1, 64))


def _weights_apply(se, so, me, mo, parts, gamma_e, gamma_o, beta_e, beta_o,
                   batch, bb):
    b, l2 = se.shape
    g = b // bb
    npart = parts.shape[0]

    def body(se_ref, so_ref, me_ref, mo_ref, p_ref, ge_ref, go_ref, be_ref,
             bo_ref, we_ref, wo_ref):
        p = jnp.sum(p_ref[...], axis=0)
        mean_e = p[0:1] / batch
        msq_e = p[1:2] / batch
        mean_o = p[2:3] / batch
        msq_o = p[3:4] / batch
        var_e = msq_e - mean_e * mean_e
        var_o = msq_o - mean_o * mean_o
        xe = ge_ref[...] * (se_ref[...] - mean_e) / jnp.sqrt(var_e + BN_EPS) \
            + be_ref[...]
        xo = go_ref[...] * (so_ref[...] - mean_o) / jnp.sqrt(var_o + BN_EPS) \
            + bo_ref[...]
        we_ref[...] = jax.nn.sigmoid(xe) * me_ref[...]
        wo_ref[...] = jax.nn.sigmoid(xo) * mo_ref[...]

    return pl.pallas_call(
        body,
        grid=(g,),
        in_specs=[
            pl.BlockSpec((bb, l2), lambda i: (i, 0)),
            pl.BlockSpec((bb, l2), lambda i: (i, 0)),
            pl.BlockSpec((bb, l2), lambda i: (i, 0)),
            pl.BlockSpec((bb, l2), lambda i: (i, 0)),
            pl.BlockSpec((npart, 4, l2), lambda i: (0, 0, 0)),
            pl.BlockSpec((1, l2), lambda i: (0, 0)),
            pl.BlockSpec((1, l2), lambda i: (0, 0)),
            pl.BlockSpec((1, l2), lambda i: (0, 0)),
            pl.BlockSpec((1, l2), lambda i: (0, 0)),
        ],
        out_specs=[
            pl.BlockSpec((bb, l2), lambda i: (i, 0)),
            pl.BlockSpec((bb, l2), lambda i: (i, 0)),
        ],
        out_shape=[
            jax.ShapeDtypeStruct((b, l2), jnp.float32),
            jax.ShapeDtypeStruct((b, l2), jnp.float32),
        ],
    )(se, so, me, mo, parts, gamma_e.reshape(1, l2), gamma_o.reshape(1, l2),
      beta_e.reshape(1, l2), beta_o.reshape(1, l2))


def _pool_tc(emb2, we, wo, bb, l2):
    ee = emb2.shape[1]
    b = emb2.shape[0] // l2
    g = b // bb

    def body(e_ref, we_ref, wo_ref, o_ref):
        e = e_ref[...].astype(jnp.float32).reshape(bb, l2, ee)
        lo = e[:, :, :64]
        hi = e[:, :, 64:]
        we = we_ref[...]
        wo = wo_ref[...]
        pe = jnp.sum(we[:, :, None] * lo, axis=1)
        po = jnp.sum(wo[:, :, None] * hi, axis=1)
        wsum = jnp.sum(we, axis=1, keepdims=True) \
            + jnp.sum(wo, axis=1, keepdims=True)
        o_ref[...] = (pe + po) / (wsum + EPS)

    return pl.pallas_call(
        body,
        grid=(g,),
        in_specs=[
            pl.BlockSpec((bb * l2, ee), lambda i: (i, 0)),
            pl.BlockSpec((bb, l2), lambda i: (i, 0)),
            pl.BlockSpec((bb, l2), lambda i: (i, 0)),
        ],
        out_specs=pl.BlockSpec((bb, 64), lambda i: (i, 0)),
        out_shape=jax.ShapeDtypeStruct((b, 64), jnp.float32),
    )(emb2, we, wo)


def _similarity(pq, pc):
    b, e = pq.shape
    ti, tj = 256, 2048

    def body(q_ref, c_ref, o_ref):
        q = q_ref[...]
        c = c_ref[...]
        qn = q / (jnp.sqrt(jnp.sum(q * q, axis=1, keepdims=True)) + EPS)
        cn = c / (jnp.sqrt(jnp.sum(c * c, axis=1, keepdims=True)) + EPS)
        o_ref[...] = lax.dot_general(
            qn, cn, (((1,), (1,)), ((), ())),
            preferred_element_type=jnp.float32)

    return pl.pallas_call(
        body,
        grid=(b // ti, b // tj),
        in_specs=[
            pl.BlockSpec((ti, e), lambda i, j: (i, 0)),
            pl.BlockSpec((tj, e), lambda i, j: (j, 0)),
        ],
        out_specs=pl.BlockSpec((ti, tj), lambda i, j: (i, j)),
        out_shape=jax.ShapeDtypeStruct((b, b), jnp.float32),
    )(pq, pc)


def _encode(seqs, table, w, gamma, beta, rows_per_block):
    b, l = seqs.shape
    lp = -(-l // 32) * 32
    if lp != l:
        seqs = jnp.pad(seqs, ((0, 0), (0, lp - l)))
        gamma = jnp.pad(gamma, (0, lp - l))
        beta = jnp.pad(beta, (0, lp - l))
    l2 = lp // 2
    tab16 = _to_bf16_linear(table, rows_per_block)
    emb = _gather_rows(tab16, seqs)
    emb3 = emb.reshape(b * l2, 128)
    se, so, me, mo, parts = _emb_scores(emb3, w, 128, l2)
    we, wo = _weights_apply(
        se, so, me, mo, parts, gamma[0::2], gamma[1::2], beta[0::2],
        beta[1::2], float(b), 128)
    return _pool_tc(emb3, we, wo, 128, l2)


def kernel(code_seqs, query_seqs, code_table, code_w, code_gamma, code_beta,
           query_table, query_w, query_gamma, query_beta):
    pq = _encode(query_seqs, query_table, query_w, query_gamma, query_beta,
                 4000)
    pc = _encode(code_seqs, code_table, code_w, code_gamma, code_beta, 8000)
    return _similarity(pq, pc)


# speedup vs baseline: 1.2825x; 1.2825x over previous
import functools

import jax
import jax.numpy as jnp
from jax import lax
from jax.experimental import pallas as pl
from jax.experimental.pallas import tpu as pltpu
from jax.experimental.pallas import tpu_sc as plsc

NC, NS = 2, 16
NW = NC * NS
EPS = 1e-8
BN_EPS = 1e-5


def _to_bf16_linear(table, rows_per_block):
    n, e = table.shape

    def body(t_ref, o_ref):
        o_ref[...] = t_ref[...].astype(jnp.bfloat16)

    return pl.pallas_call(
        body,
        grid=(n // rows_per_block,),
        in_specs=[pl.BlockSpec((rows_per_block, e), lambda i: (i, 0))],
        out_specs=pl.BlockSpec((rows_per_block, e), lambda i: (i, 0)),
        out_shape=jax.ShapeDtypeStruct((n, e), jnp.bfloat16),
    )(table)


def _gather_rows(tab16, seqs):
    b, l = seqs.shape
    n, e = tab16.shape
    bpw = b // NW
    nbuf = 4
    mesh = plsc.VectorSubcoreMesh(core_axis_name="c", subcore_axis_name="s")

    @functools.partial(
        pl.kernel,
        out_type=jax.ShapeDtypeStruct((b, l, e), jnp.bfloat16),
        mesh=mesh,
        scratch_types=[
            pltpu.VMEM((bpw, l), jnp.int32),
            pltpu.VMEM((nbuf, l, e), jnp.bfloat16),
            pltpu.SemaphoreType.DMA((nbuf,)),
            pltpu.SemaphoreType.DMA((nbuf,)),
        ],
        compiler_params=pltpu.CompilerParams(use_tc_tiling_on_sc=False),
    )
    def kern(tab_hbm, seq_hbm, out_hbm, idx_v, rows_v, gsems, osems):
        wid = lax.axis_index("s") * NC + lax.axis_index("c")
        base = wid * bpw
        pltpu.sync_copy(seq_hbm.at[pl.ds(base, bpw)], idx_v)

        def fire(g, slot):
            pltpu.async_copy(
                tab_hbm.at[idx_v.at[g]], rows_v.at[slot], gsems.at[slot])

        def gwait(g, slot):
            pltpu.make_async_copy(
                tab_hbm.at[idx_v.at[g]], rows_v.at[slot], gsems.at[slot]).wait()

        def odst(g):
            return out_hbm.at[base + g]

        def owait(g, slot):
            pltpu.make_async_copy(
                rows_v.at[slot], odst(g), osems.at[slot]).wait()

        for p in range(nbuf - 1):
            fire(p, p)

        def body(g, carry):
            slot = lax.rem(g, nbuf)

            @pl.when(g + nbuf - 1 < bpw)
            def _():
                ns = lax.rem(g + nbuf - 1, nbuf)

                @pl.when(g >= 1)
                def _():
                    owait(g - 1, ns)

                fire(g + nbuf - 1, ns)

            gwait(g, slot)
            pltpu.async_copy(rows_v.at[slot], odst(g), osems.at[slot])
            return carry

        lax.fori_loop(0, bpw, body, 0)
        for t in range(nbuf):
            g = bpw - nbuf + t
            owait(g, lax.rem(g, nbuf))

    return kern(tab16, seqs)


def _emb_scores(emb3, w, bb, l2):
    b, _, ee = emb3.shape
    g = b // bb

    def body(e_ref, w_ref, se_ref, so_ref, me_ref, mo_ref, p_ref):
        e = e_ref[...].astype(jnp.float32)
        lo = e[:, :, :64]
        hi = e[:, :, 64:]
        wv = w_ref[0]
        dn = (((2,), (0,)), ((), ()))
        se = lax.dot_general(lo, wv, dn, preferred_element_type=jnp.float32)
        so = lax.dot_general(hi, wv, dn, preferred_element_type=jnp.float32)
        se_ref[...] = se
        so_ref[...] = so
        me_ref[...] = jnp.any(lo != 0.0, axis=2).astype(jnp.float32)
        mo_ref[...] = jnp.any(hi != 0.0, axis=2).astype(jnp.float32)
        parts = jnp.concatenate(
            [jnp.sum(se, axis=0, keepdims=True),
             jnp.sum(se * se, axis=0, keepdims=True),
             jnp.sum(so, axis=0, keepdims=True),
             jnp.sum(so * so, axis=0, keepdims=True)], axis=0)
        p_ref[...] = parts.reshape(1, 4, l2)

    return pl.pallas_call(
        body,
        grid=(g,),
        in_specs=[
            pl.BlockSpec((bb, l2, ee), lambda i: (i, 0, 0)),
            pl.BlockSpec((1, 64), lambda i: (0, 0)),
        ],
        out_specs=[
            pl.BlockSpec((bb, l2), lambda i: (i, 0)),
            pl.BlockSpec((bb, l2), lambda i: (i, 0)),
            pl.BlockSpec((bb, l2), lambda i: (i, 0)),
            pl.BlockSpec((bb, l2), lambda i: (i, 0)),
            pl.BlockSpec((1, 4, l2), lambda i: (i, 0, 0)),
        ],
        out_shape=[
            jax.ShapeDtypeStruct((b, l2), jnp.float32),
            jax.ShapeDtypeStruct((b, l2), jnp.float32),
            jax.ShapeDtypeStruct((b, l2), jnp.float32),
            jax.ShapeDtypeStruct((b, l2), jnp.float32),
            jax.ShapeDtypeStruct((g, 4, l2), jnp.float32),
        ],
    )(emb3, w.reshape(1, 64))


def _weights_apply(se, so, me, mo, parts, gamma_e, gamma_o, beta_e, beta_o,
                   batch, bb):
    b, l2 = se.shape
    g = b // bb
    npart = parts.shape[0]

    def body(se_ref, so_ref, me_ref, mo_ref, p_ref, ge_ref, go_ref, be_ref,
             bo_ref, we_ref, wo_ref):
        p = jnp.sum(p_ref[...], axis=0)
        mean_e = p[0:1] / batch
        msq_e = p[1:2] / batch
        mean_o = p[2:3] / batch
        msq_o = p[3:4] / batch
        var_e = msq_e - mean_e * mean_e
        var_o = msq_o - mean_o * mean_o
        xe = ge_ref[...] * (se_ref[...] - mean_e) / jnp.sqrt(var_e + BN_EPS) \
            + be_ref[...]
        xo = go_ref[...] * (so_ref[...] - mean_o) / jnp.sqrt(var_o + BN_EPS) \
            + bo_ref[...]
        we_ref[...] = jax.nn.sigmoid(xe) * me_ref[...]
        wo_ref[...] = jax.nn.sigmoid(xo) * mo_ref[...]

    return pl.pallas_call(
        body,
        grid=(g,),
        in_specs=[
            pl.BlockSpec((bb, l2), lambda i: (i, 0)),
            pl.BlockSpec((bb, l2), lambda i: (i, 0)),
            pl.BlockSpec((bb, l2), lambda i: (i, 0)),
            pl.BlockSpec((bb, l2), lambda i: (i, 0)),
            pl.BlockSpec((npart, 4, l2), lambda i: (0, 0, 0)),
            pl.BlockSpec((1, l2), lambda i: (0, 0)),
            pl.BlockSpec((1, l2), lambda i: (0, 0)),
            pl.BlockSpec((1, l2), lambda i: (0, 0)),
            pl.BlockSpec((1, l2), lambda i: (0, 0)),
        ],
        out_specs=[
            pl.BlockSpec((bb, l2), lambda i: (i, 0)),
            pl.BlockSpec((bb, l2), lambda i: (i, 0)),
        ],
        out_shape=[
            jax.ShapeDtypeStruct((b, l2), jnp.float32),
            jax.ShapeDtypeStruct((b, l2), jnp.float32),
        ],
    )(se, so, me, mo, parts, gamma_e.reshape(1, l2), gamma_o.reshape(1, l2),
      beta_e.reshape(1, l2), beta_o.reshape(1, l2))


def _pool_tc(emb3, we, wo, bb, l2):
    b, _, ee = emb3.shape
    g = b // bb

    def body(e_ref, we_ref, wo_ref, o_ref):
        e = e_ref[...].astype(jnp.float32)
        lo = e[:, :, :64]
        hi = e[:, :, 64:]
        we = we_ref[...]
        wo = wo_ref[...]
        pe = jnp.sum(we[:, :, None] * lo, axis=1)
        po = jnp.sum(wo[:, :, None] * hi, axis=1)
        wsum = jnp.sum(we, axis=1, keepdims=True) \
            + jnp.sum(wo, axis=1, keepdims=True)
        o_ref[...] = (pe + po) / (wsum + EPS)

    return pl.pallas_call(
        body,
        grid=(g,),
        in_specs=[
            pl.BlockSpec((bb, l2, ee), lambda i: (i, 0, 0)),
            pl.BlockSpec((bb, l2), lambda i: (i, 0)),
            pl.BlockSpec((bb, l2), lambda i: (i, 0)),
        ],
        out_specs=pl.BlockSpec((bb, 64), lambda i: (i, 0)),
        out_shape=jax.ShapeDtypeStruct((b, 64), jnp.float32),
    )(emb3, we, wo)


def _similarity(pq, pc):
    b, e = pq.shape
    ti, tj = 256, 2048

    def body(q_ref, c_ref, o_ref):
        q = q_ref[...]
        c = c_ref[...]
        qn = q / (jnp.sqrt(jnp.sum(q * q, axis=1, keepdims=True)) + EPS)
        cn = c / (jnp.sqrt(jnp.sum(c * c, axis=1, keepdims=True)) + EPS)
        o_ref[...] = lax.dot_general(
            qn, cn, (((1,), (1,)), ((), ())),
            preferred_element_type=jnp.float32)

    return pl.pallas_call(
        body,
        grid=(b // ti, b // tj),
        in_specs=[
            pl.BlockSpec((ti, e), lambda i, j: (i, 0)),
            pl.BlockSpec((tj, e), lambda i, j: (j, 0)),
        ],
        out_specs=pl.BlockSpec((ti, tj), lambda i, j: (i, j)),
        out_shape=jax.ShapeDtypeStruct((b, b), jnp.float32),
    )(pq, pc)


def _encode(seqs, table, w, gamma, beta, rows_per_block):
    b, l = seqs.shape
    l2 = l // 2
    tab16 = _to_bf16_linear(table, rows_per_block)
    emb = _gather_rows(tab16, seqs)
    emb3 = emb.reshape(b, l2, 128)
    se, so, me, mo, parts = _emb_scores(emb3, w, 128, l2)
    we, wo = _weights_apply(
        se, so, me, mo, parts, gamma[0::2], gamma[1::2], beta[0::2],
        beta[1::2], float(b), 128)
    return _pool_tc(emb3, we, wo, 128, l2)


def kernel(code_seqs, query_seqs, code_table, code_w, code_gamma, code_beta,
           query_table, query_w, query_gamma, query_beta):
    pq = _encode(query_seqs, query_table, query_w, query_gamma, query_beta,
                 4000)
    pc = _encode(code_seqs, code_table, code_w, code_gamma, code_beta, 8000)
    return _similarity(pq, pc)


# speedup vs baseline: 1.3171x; 1.0269x over previous
import functools

import jax
import jax.numpy as jnp
from jax import lax
from jax.experimental import pallas as pl
from jax.experimental.pallas import tpu as pltpu
from jax.experimental.pallas import tpu_sc as plsc

NC, NS = 2, 16
NW = NC * NS
EPS = 1e-8
BN_EPS = 1e-5


def _row_scores(table, w, rows_per_block):
    n, e = table.shape
    grid = n // rows_per_block
    sub = rows_per_block // 8

    def body(t_ref, w_ref, o_ref):
        t = t_ref[...].reshape(sub, 8, e)
        o_ref[...] = lax.dot_general(
            t, w_ref[0],
            (((2,), (0,)), ((), ())),
            preferred_element_type=jnp.float32)

    out = pl.pallas_call(
        body,
        grid=(grid,),
        in_specs=[
            pl.BlockSpec((rows_per_block, e), lambda i: (i, 0)),
            pl.BlockSpec((1, e), lambda i: (0, 0)),
        ],
        out_specs=pl.BlockSpec((sub, 8), lambda i: (i, 0)),
        out_shape=jax.ShapeDtypeStruct((n // 8, 8), jnp.float32),
    )(table, w.reshape(1, e))
    return out.reshape(n)


def _row_scores_3d(table, w, rows_per_block):
    n, e = table.shape
    s = 125
    r = n // s
    t3 = table.reshape(r, s, e)

    def body(t_ref, w_ref, o_ref):
        o_ref[...] = lax.dot_general(
            t_ref[...], w_ref[0],
            (((2,), (0,)), ((), ())),
            preferred_element_type=jnp.float32)

    out = pl.pallas_call(
        body,
        grid=(r // rows_per_block,),
        in_specs=[
            pl.BlockSpec((rows_per_block, s, e), lambda i: (i, 0, 0)),
            pl.BlockSpec((1, e), lambda i: (0, 0)),
        ],
        out_specs=pl.BlockSpec((rows_per_block, s), lambda i: (i, 0)),
        out_shape=jax.ShapeDtypeStruct((r, s), jnp.float32),
    )(t3, w.reshape(1, e))
    return out.reshape(n)


def _score_gather(ts, seqs):
    b, l = seqs.shape
    t = b * l
    n = ts.shape[0]
    per_w = t // NW
    seq2 = seqs.reshape(NW, per_w)
    chunk = 10000
    nch = n // chunk
    assert nch * chunk == n
    mesh = plsc.VectorSubcoreMesh(core_axis_name="c", subcore_axis_name="s")

    @functools.partial(
        pl.kernel,
        out_type=jax.ShapeDtypeStruct((NW, per_w), jnp.float32),
        mesh=mesh,
        scratch_types=[
            pltpu.VMEM((per_w,), jnp.int32),
            pltpu.VMEM((per_w,), jnp.float32),
            pltpu.VMEM((chunk,), jnp.float32),
            pltpu.VMEM_SHARED((n,), jnp.float32),
            pltpu.SemaphoreType.DMA,
        ],
    )
    def kern(ts_hbm, seq_hbm, out_hbm, idx_v, val_v, stg_v, ts_spm, sem):
        sid = lax.axis_index("s")
        wid = sid * NC + lax.axis_index("c")

        def fill(k, carry):
            @pl.when(lax.rem(k, NS) == sid)
            def _():
                off = pl.multiple_of(k * chunk, 8)
                pltpu.sync_copy(ts_hbm.at[pl.ds(off, chunk)], stg_v)
                pltpu.sync_copy(stg_v, ts_spm.at[pl.ds(off, chunk)])
            return carry

        lax.fori_loop(0, nch, fill, 0)
        pltpu.sync_copy(seq_hbm.at[wid], idx_v)
        plsc.subcore_barrier()
        pltpu.async_copy(ts_spm.at[idx_v], val_v, sem).wait()
        pltpu.sync_copy(val_v, out_hbm.at[wid])

    return kern(ts, seq2).reshape(b, l)


def _bn_partials(scores3):
    g, bb, l = scores3.shape

    def body(s_ref, s1_ref, s2_ref):
        s = s_ref[0]
        s1_ref[...] = jnp.sum(s, axis=0).reshape(1, 1, l)
        s2_ref[...] = jnp.sum(s * s, axis=0).reshape(1, 1, l)

    return pl.pallas_call(
        body,
        grid=(g,),
        in_specs=[pl.BlockSpec((1, bb, l), lambda i: (i, 0, 0))],
        out_specs=[
            pl.BlockSpec((1, 1, l), lambda i: (i, 0, 0)),
            pl.BlockSpec((1, 1, l), lambda i: (i, 0, 0)),
        ],
        out_shape=[
            jax.ShapeDtypeStruct((g, 1, l), jnp.float32),
            jax.ShapeDtypeStruct((g, 1, l), jnp.float32),
        ],
    )(scores3)


def _weights_apply(scores3, seqs3, s1, s2, gamma, beta, batch):
    g, bb, l = scores3.shape

    def body(s_ref, q_ref, s1_ref, s2_ref, g_ref, bt_ref, o_ref):
        mean = jnp.sum(s1_ref[...], axis=0) / batch
        msq = jnp.sum(s2_ref[...], axis=0) / batch
        var = msq - mean * mean
        s = s_ref[0]
        xn = g_ref[...] * (s - mean) / jnp.sqrt(var + BN_EPS) + bt_ref[...]
        mask = (q_ref[0] != 0).astype(jnp.float32)
        o_ref[...] = (jax.nn.sigmoid(xn) * mask).reshape(1, bb, l)

    return pl.pallas_call(
        body,
        grid=(g,),
        in_specs=[
            pl.BlockSpec((1, bb, l), lambda i: (i, 0, 0)),
            pl.BlockSpec((1, bb, l), lambda i: (i, 0, 0)),
            pl.BlockSpec((g, 1, l), lambda i: (0, 0, 0)),
            pl.BlockSpec((g, 1, l), lambda i: (0, 0, 0)),
            pl.BlockSpec((1, l), lambda i: (0, 0)),
            pl.BlockSpec((1, l), lambda i: (0, 0)),
        ],
        out_specs=pl.BlockSpec((1, bb, l), lambda i: (i, 0, 0)),
        out_shape=jax.ShapeDtypeStruct((g, bb, l), jnp.float32),
    )(scores3, seqs3, s1, s2, gamma.reshape(1, l), beta.reshape(1, l))


def _pool(table, seqs, weights):
    b, l = seqs.shape
    n, e = table.shape
    bpw = b // NW
    lp = -(-l // 16) * 16
    if lp != l:
        seqs = jnp.pad(seqs, ((0, 0), (0, lp - l)))
        weights = jnp.pad(weights, ((0, 0), (0, lp - l)))
    nbuf = 4
    seq3 = seqs.reshape(NW, bpw, lp)
    w3 = weights.reshape(NW, bpw, lp)
    mesh = plsc.VectorSubcoreMesh(core_axis_name="c", subcore_axis_name="s")

    @functools.partial(
        pl.kernel,
        out_type=jax.ShapeDtypeStruct((NW, bpw, e), jnp.float32),
        mesh=mesh,
        scratch_types=[
            pltpu.VMEM((bpw, lp), jnp.int32),
            pltpu.VMEM((bpw, lp), jnp.float32),
            pltpu.VMEM((nbuf, lp, e), jnp.float32),
            pltpu.VMEM((bpw, e), jnp.float32),
            pltpu.SemaphoreType.DMA((nbuf,)),
        ],
        compiler_params=pltpu.CompilerParams(use_tc_tiling_on_sc=False),
    )
    def kern(tab_hbm, seq_hbm, w_hbm, out_hbm, idx_v, wgt_v, rows_v, out_v, sems):
        wid = lax.axis_index("s") * NC + lax.axis_index("c")
        pltpu.sync_copy(seq_hbm.at[wid], idx_v)
        pltpu.sync_copy(w_hbm.at[wid], wgt_v)

        def fire(bi, slot):
            pltpu.async_copy(
                tab_hbm.at[idx_v.at[bi]], rows_v.at[slot], sems.at[slot])

        def wait(bi, slot):
            pltpu.make_async_copy(
                tab_hbm.at[idx_v.at[bi]], rows_v.at[slot], sems.at[slot]).wait()

        for p in range(nbuf - 1):
            fire(p, p)

        def body(bi, carry):
            slot = lax.rem(bi, nbuf)

            @pl.when(bi + nbuf - 1 < bpw)
            def _():
                fire(bi + nbuf - 1, lax.rem(bi + nbuf - 1, nbuf))

            wait(bi, slot)

            zero = jnp.zeros((16,), jnp.float32)

            def inner(lg, acc):
                a0, a1, a2, a3, wsv = acc
                wvec = wgt_v[bi, pl.ds(lg * 16, 16)]
                for kk in range(16):
                    wv = wvec[kk]
                    li = lg * 16 + kk
                    a0 = a0 + wv * rows_v[slot, li, pl.ds(0, 16)]
                    a1 = a1 + wv * rows_v[slot, li, pl.ds(16, 16)]
                    a2 = a2 + wv * rows_v[slot, li, pl.ds(32, 16)]
                    a3 = a3 + wv * rows_v[slot, li, pl.ds(48, 16)]
                return (a0, a1, a2, a3, wsv + wvec)

            a0, a1, a2, a3, wsv = lax.fori_loop(
                0, lp // 16, inner, (zero, zero, zero, zero, zero))
            ws = wsv[0]
            for kk in range(1, 16):
                ws = ws + wsv[kk]
            d = ws + EPS
            out_v[bi, pl.ds(0, 16)] = a0 / d
            out_v[bi, pl.ds(16, 16)] = a1 / d
            out_v[bi, pl.ds(32, 16)] = a2 / d
            out_v[bi, pl.ds(48, 16)] = a3 / d
            return carry

        lax.fori_loop(0, bpw, body, 0)
        pltpu.sync_copy(out_v, out_hbm.at[wid])

    return kern(table, seq3, w3).reshape(b, e)


def _similarity(pq, pc):
    b, e = pq.shape
    ti, tj = 256, 2048

    def body(q_ref, c_ref, o_ref):
        q = q_ref[...]
        c = c_ref[...]
        qn = q / (jnp.sqrt(jnp.sum(q * q, axis=1, keepdims=True)) + EPS)
        cn = c / (jnp.sqrt(jnp.sum(c * c, axis=1, keepdims=True)) + EPS)
        o_ref[...] = lax.dot_general(
            qn, cn, (((1,), (1,)), ((), ())),
            preferred_element_type=jnp.float32)

    return pl.pallas_call(
        body,
        grid=(b // ti, b // tj),
        in_specs=[
            pl.BlockSpec((ti, e), lambda i, j: (i, 0)),
            pl.BlockSpec((tj, e), lambda i, j: (j, 0)),
        ],
        out_specs=pl.BlockSpec((ti, tj), lambda i, j: (i, j)),
        out_shape=jax.ShapeDtypeStruct((b, b), jnp.float32),
    )(pq, pc)


def _encode(seqs, table, w, gamma, beta, rows_per_block):
    b, l = seqs.shape
    if (table.shape[0] % rows_per_block == 0
            and (rows_per_block // 8) % 8 == 0):
        ts = _row_scores(table, w, rows_per_block)
    else:
        ts = _row_scores_3d(table, w, rows_per_block)
    scores = _score_gather(ts, seqs)
    scores3 = scores.reshape(NW, b // NW, l)
    seqs3 = seqs.reshape(NW, b // NW, l)
    s1, s2 = _bn_partials(scores3)
    weights = _weights_apply(scores3, seqs3, s1, s2, gamma, beta,
                             float(b)).reshape(b, l)
    return _pool(table, seqs, weights)


def kernel(code_seqs, query_seqs, code_table, code_w, code_gamma, code_beta,
           query_table, query_w, query_gamma, query_beta):
    pq = _encode(query_seqs, query_table, query_w, query_gamma, query_beta, 80)
    pc = _encode(code_seqs, code_table, code_w, code_gamma, code_beta, 8000)
    return _similarity(pq, pc)
